# baseline (speedup 1.0000x reference)
import functools

import jax
import jax.numpy as jnp
from jax import lax
from jax.experimental import pallas as pl
from jax.experimental.pallas import tpu as pltpu
from jax.experimental.pallas import tpu_sc as plsc

_LANES = 128
_K = 5
_D = 64
_NW = 32


def _gather_body(x_hbm, table_hbm, out_hbm, idx_all, rows_v, gsem0, gsem1,
                 wsem0, wsem1, *, rows_per_w):
    nc = 2
    wid = lax.axis_index("s") * nc + lax.axis_index("c")
    row0 = wid * rows_per_w
    steps = rows_per_w // _K
    gsems = (gsem0, gsem1)
    wsems = (wsem0, wsem1)

    pltpu.sync_copy(x_hbm.at[pl.ds(row0, rows_per_w), :], idx_all)

    def fire(g, b):
        for j in range(_K):
            pltpu.async_copy(
                table_hbm.at[idx_all.at[g * _K + j]],
                rows_v.at[b, j],
                gsems[b],
            )

    def complete(g, b):
        for j in range(_K):
            pltpu.make_async_copy(
                table_hbm.at[idx_all.at[g * _K + j]], rows_v.at[b, j], gsems[b]
            ).wait()
        pltpu.async_copy(
            rows_v.at[b], out_hbm.at[pl.ds(row0 + g * _K, _K)], wsems[b]
        )

    def wait_wb(g, b):
        pltpu.make_async_copy(
            rows_v.at[b], out_hbm.at[pl.ds(row0 + g * _K, _K)], wsems[b]
        ).wait()

    fire(0, 0)
    fire(1, 1)

    def pair(p, carry):
        g0 = 2 * p
        for b in (0, 1):
            g = g0 + b
            complete(g, b)
            wait_wb(g, b)
            fire(g + 2, b)
        return carry

    lax.fori_loop(0, steps // 2 - 1, pair, 0)

    complete(steps - 2, 0)
    complete(steps - 1, 1)
    wait_wb(steps - 2, 0)
    wait_wb(steps - 1, 1)


def kernel(x, table):
    b, l = x.shape
    n = b * l
    assert n % _LANES == 0
    nrows = n // _LANES
    assert nrows % (_NW * _K * 2) == 0
    rows_per_w = nrows // _NW

    x2 = x.reshape(nrows, _LANES)

    mesh = plsc.VectorSubcoreMesh(core_axis_name="c", subcore_axis_name="s")
    gather = functools.partial(
        pl.kernel,
        mesh=mesh,
        out_type=jax.ShapeDtypeStruct((nrows, _LANES, _D), jnp.float32),
        scratch_types=[
            pltpu.VMEM((rows_per_w, _LANES), jnp.int32),
            pltpu.VMEM((2, _K, _LANES, _D), jnp.float32),
            pltpu.SemaphoreType.DMA,
            pltpu.SemaphoreType.DMA,
            pltpu.SemaphoreType.DMA,
            pltpu.SemaphoreType.DMA,
        ],
        compiler_params=pltpu.CompilerParams(use_tc_tiling_on_sc=False),
    )(functools.partial(_gather_body, rows_per_w=rows_per_w))

    out = gather(x2, table)
    return out.reshape(b, l, _D)


# speedup vs baseline: 1.0016x; 1.0016x over previous
import functools

import jax
import jax.numpy as jnp
from jax import lax
from jax.experimental import pallas as pl
from jax.experimental.pallas import tpu as pltpu
from jax.experimental.pallas import tpu_sc as plsc

_K = 4
_NW = 32
_SPLITS = ((0, 128), (128, 72))


def _gather_body(x_hbm, table_hbm, out_hbm, idx_all, rows_v, gsem0, gsem1,
                 wsem0, wsem1, *, rows_per_w, seq_len):
    nc = 2
    wid = lax.axis_index("s") * nc + lax.axis_index("c")
    row0 = wid * rows_per_w
    steps = rows_per_w // _K
    gsems = (gsem0, gsem1)
    wsems = (wsem0, wsem1)

    pltpu.sync_copy(x_hbm.at[pl.ds(row0, rows_per_w), :], idx_all)

    def fire(g, b):
        for kr in range(_K):
            for off, ln in _SPLITS:
                pltpu.async_copy(
                    table_hbm.at[idx_all.at[g * _K + kr, pl.ds(off, ln)]],
                    rows_v.at[b, kr, pl.ds(off, ln)],
                    gsems[b],
                )

    def complete(g, b):
        for kr in range(_K):
            for off, ln in _SPLITS:
                pltpu.make_async_copy(
                    table_hbm.at[idx_all.at[g * _K + kr, pl.ds(off, ln)]],
                    rows_v.at[b, kr, pl.ds(off, ln)],
                    gsems[b],
                ).wait()
        pltpu.async_copy(
            rows_v.at[b], out_hbm.at[pl.ds(row0 + g * _K, _K)], wsems[b]
        )

    def wait_wb(g, b):
        pltpu.make_async_copy(
            rows_v.at[b], out_hbm.at[pl.ds(row0 + g * _K, _K)], wsems[b]
        ).wait()

    fire(0, 0)
    fire(1, 1)

    def pair(p, carry):
        g0 = 2 * p
        for b in (0, 1):
            g = g0 + b
            complete(g, b)
            wait_wb(g, b)
            fire(g + 2, b)
        return carry

    lax.fori_loop(0, steps // 2 - 1, pair, 0)

    complete(steps - 2, 0)
    complete(steps - 1, 1)
    wait_wb(steps - 2, 0)
    wait_wb(steps - 1, 1)


def kernel(x, table):
    b, l = x.shape
    d = table.shape[1]
    assert b % (_NW * _K * 2) == 0
    rows_per_w = b // _NW

    mesh = plsc.VectorSubcoreMesh(core_axis_name="c", subcore_axis_name="s")
    gather = functools.partial(
        pl.kernel,
        mesh=mesh,
        out_type=jax.ShapeDtypeStruct((b, l, d), jnp.float32),
        scratch_types=[
            pltpu.VMEM((rows_per_w, l), jnp.int32),
            pltpu.VMEM((2, _K, l, d), jnp.float32),
            pltpu.SemaphoreType.DMA,
            pltpu.SemaphoreType.DMA,
            pltpu.SemaphoreType.DMA,
            pltpu.SemaphoreType.DMA,
        ],
        compiler_params=pltpu.CompilerParams(use_tc_tiling_on_sc=False),
    )(functools.partial(_gather_body, rows_per_w=rows_per_w, seq_len=l))

    return gather(x, table)
